# baseline (speedup 1.0000x reference)
import functools

import jax
import jax.numpy as jnp
from jax.experimental import pallas as pl

_EPS = 0.1
_IGNORE = 0


def _ce_body(x_ref, lab_ref, num_ref, den_ref):
    i = pl.program_id(0)
    x = x_ref[...]
    lab = lab_ref[0, 0, :]
    k = x.shape[1]

    m = jnp.max(x, axis=1, keepdims=True)
    s = jnp.sum(jnp.exp(x - m), axis=1, keepdims=True)
    t = jnp.sum(x, axis=1, keepdims=True)
    lse = m + jnp.log(s)

    cols = jax.lax.broadcasted_iota(jnp.int32, x.shape, 1)
    g = jnp.sum(jnp.where(cols == lab[:, None], x, 0.0), axis=1,
                keepdims=True)

    mask = (lab[:, None] != _IGNORE).astype(x.dtype)
    per = lse - (1.0 - _EPS) * g - (_EPS / k) * t
    pnum = jnp.sum(per * mask, axis=0, keepdims=True)
    pden = jnp.sum(mask, axis=0, keepdims=True)

    @pl.when(i == 0)
    def _init():
        num_ref[...] = jnp.zeros_like(num_ref)
        den_ref[...] = jnp.zeros_like(den_ref)

    num_ref[...] += pnum
    den_ref[...] += pden


@functools.partial(jax.jit, static_argnames=("rb",))
def _ce_loss(preds, labels, rb=8):
    r, k = preds.shape
    nb = r // rb
    lab3 = labels.astype(jnp.int32).reshape(nb, 1, rb)
    num, den = pl.pallas_call(
        _ce_body,
        grid=(nb,),
        in_specs=[
            pl.BlockSpec((rb, k), lambda i: (i, 0)),
            pl.BlockSpec((1, 1, rb), lambda i: (i, 0, 0)),
        ],
        out_specs=[
            pl.BlockSpec((1, 1), lambda i: (0, 0)),
            pl.BlockSpec((1, 1), lambda i: (0, 0)),
        ],
        out_shape=[
            jax.ShapeDtypeStruct((1, 1), preds.dtype),
            jax.ShapeDtypeStruct((1, 1), preds.dtype),
        ],
    )(preds, lab3)
    return num[0, 0] / den[0, 0]


def kernel(preds, labels):
    return _ce_loss(preds, labels)


# speedup vs baseline: 1.2290x; 1.2290x over previous
import functools

import jax
import jax.numpy as jnp
from jax.experimental import pallas as pl

_EPS = 0.1
_IGNORE = 0


def _ce_body(x_ref, lab_ref, num_ref, den_ref):
    i = pl.program_id(0)
    x = x_ref[...]
    lab = lab_ref[0, 0, :]
    k = x.shape[1]

    m = jnp.max(x, axis=1, keepdims=True)
    s = jnp.sum(jnp.exp(x - m), axis=1, keepdims=True)
    lse = m + jnp.log(s)

    cols = jax.lax.broadcasted_iota(jnp.int32, x.shape, 1)
    w = jnp.where(cols == lab[:, None], (1.0 - _EPS) + _EPS / k, _EPS / k)
    wx = jnp.sum(w * x, axis=1, keepdims=True)

    mask = (lab[:, None] != _IGNORE).astype(x.dtype)
    per = lse - wx
    pnum = jnp.sum(per * mask, axis=0, keepdims=True)
    pden = jnp.sum(mask, axis=0, keepdims=True)

    @pl.when(i == 0)
    def _init():
        num_ref[...] = jnp.zeros_like(num_ref)
        den_ref[...] = jnp.zeros_like(den_ref)

    num_ref[...] += pnum
    den_ref[...] += pden


@functools.partial(jax.jit, static_argnames=("rb",))
def _ce_loss(preds, labels, rb=32):
    r, k = preds.shape
    nb = r // rb
    lab3 = labels.astype(jnp.int32).reshape(nb, 1, rb)
    num, den = pl.pallas_call(
        _ce_body,
        grid=(nb,),
        in_specs=[
            pl.BlockSpec((rb, k), lambda i: (i, 0)),
            pl.BlockSpec((1, 1, rb), lambda i: (i, 0, 0)),
        ],
        out_specs=[
            pl.BlockSpec((1, 1), lambda i: (0, 0)),
            pl.BlockSpec((1, 1), lambda i: (0, 0)),
        ],
        out_shape=[
            jax.ShapeDtypeStruct((1, 1), preds.dtype),
            jax.ShapeDtypeStruct((1, 1), preds.dtype),
        ],
    )(preds, lab3)
    return num[0, 0] / den[0, 0]


def kernel(preds, labels):
    return _ce_loss(preds, labels)
